# Initial kernel scaffold; baseline (speedup 1.0000x reference)
import jax
import jax.numpy as jnp
from jax.experimental import pallas as pl


def kernel(x, pos, edge_index, edge_attr, batch, W_in, b_in, msg_W1, msg_b1, msg_W2, msg_b2, upd_W1, upd_b1, upd_W2, upd_b2, head_e_W, head_e_b, head_i_W, head_i_b):
    raise NotImplementedError("write your pallas kernel here")


# speedup vs baseline: 1.8493x; 1.8493x over previous
import functools

import jax
import jax.numpy as jnp
from jax import lax
from jax.experimental import pallas as pl
from jax.experimental.pallas import tpu as pltpu
from jax.experimental.pallas import tpu_sc as plsc

N = 50000
E = 800000
D = 64
G = 256
NLAYERS = 4

NC = 2
NS = 16
NW = NC * NS
CH = 128
NCHT = 200
EPAD = NW * CH * NCHT
ROWS2D = EPAD // CH
BE = 2048
BN = 1000
HALF = N // 2
HT = 25088
GARB = HALF
TPS = HT // NS
SC_NCH = ROWS2D // NS


def _sc_mesh():
    return plsc.VectorSubcoreMesh(core_axis_name="c", subcore_axis_name="s",
                                  num_cores=NC, num_subcores=NS)


_SC_PARAMS = pltpu.CompilerParams(use_tc_tiling_on_sc=False)


def _make_gather_pair(drow, interpret=False):
    @functools.partial(
        pl.kernel,
        mesh=_sc_mesh(),
        out_type=(
            jax.ShapeDtypeStruct((EPAD, drow), jnp.float32),
            jax.ShapeDtypeStruct((EPAD, drow), jnp.float32),
        ),
        scratch_types=[
            pltpu.VMEM((NCHT, CH), jnp.int32),
            pltpu.VMEM((NCHT, CH), jnp.int32),
            pltpu.VMEM((2, CH, drow), jnp.float32),
            pltpu.VMEM((2, CH, drow), jnp.float32),
            pltpu.SemaphoreType.DMA,
            pltpu.SemaphoreType.DMA,
            pltpu.SemaphoreType.DMA,
            pltpu.SemaphoreType.DMA,
        ],
        compiler_params=_SC_PARAMS,
        interpret=interpret,
    )
    def k(ta, tb, idxd, idxs, outd, outs, idxd_v, idxs_v, bufa, bufb,
          sa0, sa1, sb0, sb1):
        wid = lax.axis_index("s") * NC + lax.axis_index("c")
        r0 = wid * NCHT
        pltpu.sync_copy(idxd.at[pl.ds(r0, NCHT)], idxd_v)
        pltpu.sync_copy(idxs.at[pl.ds(r0, NCHT)], idxs_v)
        sems = ((sa0, sb0), (sa1, sb1))

        def start(j, b):
            pltpu.async_copy(ta.at[idxd_v.at[j]], bufa.at[b], sems[b][0])
            pltpu.async_copy(tb.at[idxs_v.at[j]], bufb.at[b], sems[b][1])

        def drain(j, b):
            base = (r0 + j) * CH
            pltpu.make_async_copy(ta.at[idxd_v.at[j]], bufa.at[b],
                                  sems[b][0]).wait()
            pltpu.sync_copy(bufa.at[b], outd.at[pl.ds(base, CH)])
            pltpu.make_async_copy(tb.at[idxs_v.at[j]], bufb.at[b],
                                  sems[b][1]).wait()
            pltpu.sync_copy(bufb.at[b], outs.at[pl.ds(base, CH)])

        start(0, 0)

        def body(g, carry):
            j = 2 * g
            start(j + 1, 1)
            drain(j, 0)

            @pl.when(j + 2 < NCHT)
            def _():
                start(j + 2, 0)

            drain(j + 1, 1)
            return carry

        lax.fori_loop(0, NCHT // 2, body, 0)

    return k


def _make_scatter(interpret=False):
    @functools.partial(
        pl.kernel,
        mesh=_sc_mesh(),
        out_type=jax.ShapeDtypeStruct((N, D), jnp.float32),
        scratch_types=[
            pltpu.VMEM_SHARED((HT, D), jnp.float32),
            pltpu.VMEM((2, CH), jnp.int32),
            pltpu.VMEM((2, CH, D), jnp.float32),
            pltpu.SemaphoreType.DMA,
            pltpu.SemaphoreType.DMA,
            pltpu.SemaphoreType.DMA,
            pltpu.SemaphoreType.DMA,
        ],
        compiler_params=_SC_PARAMS,
        interpret=interpret,
    )
    def k(m_hbm, idx0, idx1, zrows, out, table, ibuf, mbuf,
          sm0, sm1, si0, si1):
        c = lax.axis_index("c")
        s = lax.axis_index("s")
        r0 = s * SC_NCH
        pltpu.sync_copy(zrows, table.at[pl.ds(s * TPS, TPS)])
        plsc.subcore_barrier()
        sems = (sm0, sm1)
        isems = (si0, si1)

        def start(j, b):
            pltpu.async_copy(m_hbm.at[pl.ds((r0 + j) * CH, CH)], mbuf.at[b],
                             sems[b])

            @pl.when(c == 0)
            def _():
                pltpu.async_copy(idx0.at[pl.ds(r0 + j, 1)],
                                 ibuf.at[pl.ds(b, 1)], isems[b])

            @pl.when(c == 1)
            def _():
                pltpu.async_copy(idx1.at[pl.ds(r0 + j, 1)],
                                 ibuf.at[pl.ds(b, 1)], isems[b])

        def proc(j, b):
            pltpu.make_async_copy(m_hbm.at[pl.ds((r0 + j) * CH, CH)],
                                  mbuf.at[b], sems[b]).wait()
            pltpu.make_async_copy(idx0.at[pl.ds(r0 + j, 1)],
                                  ibuf.at[pl.ds(b, 1)], isems[b]).wait()
            pltpu.sync_copy(mbuf.at[b], table.at[ibuf.at[b]], add=True)

        start(0, 0)

        def body(g, carry):
            j = 2 * g
            start(j + 1, 1)
            proc(j, 0)

            @pl.when(j + 2 < SC_NCH)
            def _():
                start(j + 2, 0)

            proc(j + 1, 1)
            return carry

        lax.fori_loop(0, SC_NCH // 2, body, 0)
        plsc.subcore_barrier()
        last = HALF - 15 * TPS

        @pl.when(s < 15)
        def _():
            pltpu.sync_copy(table.at[pl.ds(s * TPS, TPS)],
                            out.at[pl.ds(c * HALF + s * TPS, TPS)])

        @pl.when(s == 15)
        def _():
            pltpu.sync_copy(table.at[pl.ds(15 * TPS, last)],
                            out.at[pl.ds(c * HALF + 15 * TPS, last)])

    return k


def _make_lin(interpret=False):
    def body(x_ref, w_ref, b_ref, o_ref):
        o_ref[...] = (
            jnp.dot(x_ref[...], w_ref[...], preferred_element_type=jnp.float32)
            + b_ref[...]
        )

    return pl.pallas_call(
        body,
        grid=(N // BN,),
        in_specs=[
            pl.BlockSpec((BN, 16), lambda i: (i, 0)),
            pl.BlockSpec((16, D), lambda i: (0, 0)),
            pl.BlockSpec((1, D), lambda i: (0, 0)),
        ],
        out_specs=pl.BlockSpec((BN, D), lambda i: (i, 0)),
        out_shape=jax.ShapeDtypeStruct((N, D), jnp.float32),
        interpret=interpret,
    )


def _make_proj(interpret=False):
    def body(h_ref, wd_ref, ws_ref, a_ref, b_ref):
        h = h_ref[...]
        a_ref[...] = jnp.dot(h, wd_ref[...], preferred_element_type=jnp.float32)
        b_ref[...] = jnp.dot(h, ws_ref[...], preferred_element_type=jnp.float32)

    return pl.pallas_call(
        body,
        grid=(N // BN,),
        in_specs=[
            pl.BlockSpec((BN, D), lambda i: (i, 0)),
            pl.BlockSpec((D, D), lambda i: (0, 0)),
            pl.BlockSpec((D, D), lambda i: (0, 0)),
        ],
        out_specs=[
            pl.BlockSpec((BN, D), lambda i: (i, 0)),
            pl.BlockSpec((BN, D), lambda i: (i, 0)),
        ],
        out_shape=[
            jax.ShapeDtypeStruct((N, D), jnp.float32),
            jax.ShapeDtypeStruct((N, D), jnp.float32),
        ],
        interpret=interpret,
    )


def _make_localidx(interpret=False):
    BR = 64

    def body(d_ref, o0_ref, o1_ref):
        i = pl.program_id(0)
        dv = d_ref[...]
        r2 = lax.broadcasted_iota(jnp.int32, (BR, CH), 0)
        c2 = lax.broadcasted_iota(jnp.int32, (BR, CH), 1)
        eid = (i * BR + r2) * CH + c2
        pad = eid >= E
        o0_ref[...] = jnp.where((dv < HALF) & ~pad, dv, GARB)
        o1_ref[...] = jnp.where((dv >= HALF) & ~pad, dv - HALF, GARB)

    return pl.pallas_call(
        body,
        grid=(ROWS2D // BR,),
        in_specs=[pl.BlockSpec((BR, CH), lambda i: (i, 0))],
        out_specs=[
            pl.BlockSpec((BR, CH), lambda i: (i, 0)),
            pl.BlockSpec((BR, CH), lambda i: (i, 0)),
        ],
        out_shape=[
            jax.ShapeDtypeStruct((ROWS2D, CH), jnp.int32),
            jax.ShapeDtypeStruct((ROWS2D, CH), jnp.int32),
        ],
        interpret=interpret,
    )


def _make_edge(interpret=False):
    def body(gd_ref, gs_ref, ea_ref, pd_ref, ps_ref, w1e_ref, w1p_ref,
             b1_ref, w2_ref, b2_ref, o_ref):
        dp = pd_ref[...] - ps_ref[...]
        d2 = jnp.sum(dp * dp, axis=1, keepdims=True)
        z = (
            gd_ref[...]
            + gs_ref[...]
            + jnp.dot(ea_ref[...], w1e_ref[...],
                      preferred_element_type=jnp.float32)
            + d2 * w1p_ref[...]
            + b1_ref[...]
        )
        m = (
            jnp.dot(jnp.maximum(z, 0.0), w2_ref[...],
                    preferred_element_type=jnp.float32)
            + b2_ref[...]
        )
        o_ref[...] = m

    return pl.pallas_call(
        body,
        grid=(EPAD // BE,),
        in_specs=[
            pl.BlockSpec((BE, D), lambda i: (i, 0)),
            pl.BlockSpec((BE, D), lambda i: (i, 0)),
            pl.BlockSpec((BE, 16), lambda i: (i, 0)),
            pl.BlockSpec((BE, 16), lambda i: (i, 0)),
            pl.BlockSpec((BE, 16), lambda i: (i, 0)),
            pl.BlockSpec((16, D), lambda i: (0, 0)),
            pl.BlockSpec((1, D), lambda i: (0, 0)),
            pl.BlockSpec((1, D), lambda i: (0, 0)),
            pl.BlockSpec((D, D), lambda i: (0, 0)),
            pl.BlockSpec((1, D), lambda i: (0, 0)),
        ],
        out_specs=pl.BlockSpec((BE, D), lambda i: (i, 0)),
        out_shape=jax.ShapeDtypeStruct((EPAD, D), jnp.float32),
        interpret=interpret,
    )


def _make_update(interpret=False):
    def body(h_ref, a_ref, u1h_ref, u1a_ref, ub1_ref, u2_ref, ub2_ref, o_ref):
        u = (
            jnp.dot(h_ref[...], u1h_ref[...],
                    preferred_element_type=jnp.float32)
            + jnp.dot(a_ref[...], u1a_ref[...],
                      preferred_element_type=jnp.float32)
            + ub1_ref[...]
        )
        o_ref[...] = (
            jnp.dot(jnp.maximum(u, 0.0), u2_ref[...],
                    preferred_element_type=jnp.float32)
            + ub2_ref[...]
        )

    return pl.pallas_call(
        body,
        grid=(N // BN,),
        in_specs=[
            pl.BlockSpec((BN, D), lambda i: (i, 0)),
            pl.BlockSpec((BN, D), lambda i: (i, 0)),
            pl.BlockSpec((D, D), lambda i: (0, 0)),
            pl.BlockSpec((D, D), lambda i: (0, 0)),
            pl.BlockSpec((1, D), lambda i: (0, 0)),
            pl.BlockSpec((D, D), lambda i: (0, 0)),
            pl.BlockSpec((1, D), lambda i: (0, 0)),
        ],
        out_specs=pl.BlockSpec((BN, D), lambda i: (i, 0)),
        out_shape=jax.ShapeDtypeStruct((N, D), jnp.float32),
        interpret=interpret,
    )


def _make_pool(interpret=False):
    nblk = N // BN

    def body(h_ref, b_ref, wh_ref, bh_ref, o_ref, acc, cnt):
        i = pl.program_id(0)

        @pl.when(i == 0)
        def _():
            acc[...] = jnp.zeros_like(acc)
            cnt[...] = jnp.zeros_like(cnt)

        bt = b_ref[...].reshape(1, BN)
        gi = lax.broadcasted_iota(jnp.int32, (G, BN), 0)
        oh = (bt == gi).astype(jnp.float32)
        acc[...] += jnp.dot(oh, h_ref[...], preferred_element_type=jnp.float32)
        cnt[...] += jnp.sum(oh, axis=1, keepdims=True)

        @pl.when(i == nblk - 1)
        def _():
            pooled = acc[...] / jnp.maximum(cnt[...], 1.0)
            o_ref[...] = (
                jnp.dot(pooled, wh_ref[...], preferred_element_type=jnp.float32)
                + bh_ref[...]
            )

    return pl.pallas_call(
        body,
        grid=(nblk,),
        in_specs=[
            pl.BlockSpec((BN, D), lambda i: (i, 0)),
            pl.BlockSpec((1, 1, BN), lambda i: (i, 0, 0)),
            pl.BlockSpec((D, 2 * 300), lambda i: (0, 0)),
            pl.BlockSpec((1, 2 * 300), lambda i: (0, 0)),
        ],
        out_specs=pl.BlockSpec((G, 2 * 300), lambda i: (0, 0)),
        out_shape=jax.ShapeDtypeStruct((G, 2 * 300), jnp.float32),
        scratch_shapes=[
            pltpu.VMEM((G, D), jnp.float32),
            pltpu.VMEM((G, 1), jnp.float32),
        ],
        interpret=interpret,
    )


def _build(interpret=False, sc_interpret=False):
    fns = {
        "lin": _make_lin(interpret),
        "proj": _make_proj(interpret),
        "localidx": _make_localidx(interpret),
        "edge": _make_edge(interpret),
        "update": _make_update(interpret),
        "pool": _make_pool(interpret),
        "gather16": _make_gather_pair(16, sc_interpret),
        "gather64": _make_gather_pair(D, sc_interpret),
        "scatter": _make_scatter(sc_interpret),
    }
    return fns


@functools.lru_cache(maxsize=1)
def _default_fns():
    return _build()


def kernel(x, pos, edge_index, edge_attr, batch, W_in, b_in,
           msg_W1, msg_b1, msg_W2, msg_b2,
           upd_W1, upd_b1, upd_W2, upd_b2,
           head_e_W, head_e_b, head_i_W, head_i_b, _fns=None):
    f = _fns if _fns is not None else _default_fns()
    x16 = jnp.pad(x, ((0, 0), (0, 16 - x.shape[1])))
    win16 = jnp.pad(W_in, ((0, 16 - W_in.shape[0]), (0, 0)))
    pos16 = jnp.pad(pos, ((0, 0), (0, 16 - pos.shape[1])))
    src = jnp.pad(edge_index[0], (0, EPAD - E)).reshape(ROWS2D, CH)
    dst = jnp.pad(edge_index[1], (0, EPAD - E)).reshape(ROWS2D, CH)
    ea = jnp.pad(edge_attr, ((0, EPAD - E), (0, 0)))
    zrows = jnp.zeros((TPS, D), jnp.float32)
    batch3d = batch.reshape(N // BN, 1, BN)
    wh = jnp.concatenate([head_e_W, head_i_W], axis=1)
    bh = jnp.concatenate([head_e_b, head_i_b]).reshape(1, -1)

    h = f["lin"](x16, win16, b_in.reshape(1, D))
    pd, ps = f["gather16"](pos16, pos16, dst, src)
    idx0, idx1 = f["localidx"](dst)
    for l in range(NLAYERS):
        w1 = msg_W1[l]
        a, b = f["proj"](h, w1[:D], w1[D:2 * D])
        gd, gs = f["gather64"](a, b, dst, src)
        m = f["edge"](gd, gs, ea, pd, ps, w1[2 * D:2 * D + 16],
                      w1[2 * D + 16:], msg_b1[l].reshape(1, D),
                      msg_W2[l], msg_b2[l].reshape(1, D))
        agg = f["scatter"](m, idx0, idx1, zrows)
        h = f["update"](h, agg, upd_W1[l][:D], upd_W1[l][D:],
                        upd_b1[l].reshape(1, D), upd_W2[l],
                        upd_b2[l].reshape(1, D))
    return f["pool"](h, batch3d, wh, bh)


# speedup vs baseline: 1.9945x; 1.0785x over previous
import functools

import jax
import jax.numpy as jnp
from jax import lax
from jax.experimental import pallas as pl
from jax.experimental.pallas import tpu as pltpu
from jax.experimental.pallas import tpu_sc as plsc

N = 50000
E = 800000
D = 64
G = 256
NLAYERS = 4

NC = 2
NS = 16
NW = NC * NS
CH = 128
NCHT = 200
EPAD = NW * CH * NCHT
ROWS2D = EPAD // CH
BE = 2048
BN = 1000
HALF = N // 2
HT = 25088
GARB = HALF
TPS = HT // NS
SC_NCH = ROWS2D // NS


def _sc_mesh():
    return plsc.VectorSubcoreMesh(core_axis_name="c", subcore_axis_name="s",
                                  num_cores=NC, num_subcores=NS)


_SC_PARAMS = pltpu.CompilerParams(use_tc_tiling_on_sc=False)


def _make_gather_pair(drow, dtype=jnp.float32, interpret=False):
    @functools.partial(
        pl.kernel,
        mesh=_sc_mesh(),
        out_type=(
            jax.ShapeDtypeStruct((EPAD, drow), dtype),
            jax.ShapeDtypeStruct((EPAD, drow), dtype),
        ),
        scratch_types=[
            pltpu.VMEM((NCHT, CH), jnp.int32),
            pltpu.VMEM((NCHT, CH), jnp.int32),
            pltpu.VMEM((4, CH, drow), dtype),
            pltpu.VMEM((4, CH, drow), dtype),
            [pltpu.SemaphoreType.DMA] * 4,
            [pltpu.SemaphoreType.DMA] * 4,
        ],
        compiler_params=_SC_PARAMS,
        interpret=interpret,
    )
    def k(ta, tb, idxd, idxs, outd, outs, idxd_v, idxs_v, bufa, bufb,
          sa, sb):
        wid = lax.axis_index("s") * NC + lax.axis_index("c")
        r0 = wid * NCHT
        pltpu.sync_copy(idxd.at[pl.ds(r0, NCHT)], idxd_v)
        pltpu.sync_copy(idxs.at[pl.ds(r0, NCHT)], idxs_v)

        def start(j, b):
            pltpu.async_copy(ta.at[idxd_v.at[j]], bufa.at[b], sa[b])
            pltpu.async_copy(tb.at[idxs_v.at[j]], bufb.at[b], sb[b])

        def drain(j, b):
            base = (r0 + j) * CH
            pltpu.make_async_copy(ta.at[idxd_v.at[j]], bufa.at[b],
                                  sa[b]).wait()
            pltpu.sync_copy(bufa.at[b], outd.at[pl.ds(base, CH)])
            pltpu.make_async_copy(tb.at[idxs_v.at[j]], bufb.at[b],
                                  sb[b]).wait()
            pltpu.sync_copy(bufb.at[b], outs.at[pl.ds(base, CH)])

        for b in range(4):
            start(b, b)

        def body(g, carry):
            j = 4 * g
            for b in range(4):
                drain(j + b, b)

                @pl.when(j + b + 4 < NCHT)
                def _():
                    start(j + b + 4, b)

            return carry

        lax.fori_loop(0, NCHT // 4, body, 0)

    return k


def _make_scatter(interpret=False):
    @functools.partial(
        pl.kernel,
        mesh=_sc_mesh(),
        out_type=jax.ShapeDtypeStruct((N, D), jnp.float32),
        scratch_types=[
            pltpu.VMEM_SHARED((HT, D), jnp.float32),
            pltpu.VMEM((2, CH), jnp.int32),
            pltpu.VMEM((2, CH, D), jnp.float32),
            [pltpu.SemaphoreType.DMA] * 2,
            [pltpu.SemaphoreType.DMA] * 2,
        ],
        compiler_params=_SC_PARAMS,
        interpret=interpret,
    )
    def k(m_hbm, idx0, idx1, zrows, out, table, ibuf, mbuf, sm, si):
        c = lax.axis_index("c")
        s = lax.axis_index("s")
        r0 = s * SC_NCH
        pltpu.sync_copy(zrows, table.at[pl.ds(s * TPS, TPS)])
        plsc.subcore_barrier()

        def start(j, b):
            pltpu.async_copy(m_hbm.at[pl.ds((r0 + j) * CH, CH)], mbuf.at[b],
                             sm[b])

            @pl.when(c == 0)
            def _():
                pltpu.async_copy(idx0.at[pl.ds(r0 + j, 1)],
                                 ibuf.at[pl.ds(b, 1)], si[b])

            @pl.when(c == 1)
            def _():
                pltpu.async_copy(idx1.at[pl.ds(r0 + j, 1)],
                                 ibuf.at[pl.ds(b, 1)], si[b])

        def proc(j, b):
            pltpu.make_async_copy(m_hbm.at[pl.ds((r0 + j) * CH, CH)],
                                  mbuf.at[b], sm[b]).wait()
            pltpu.make_async_copy(idx0.at[pl.ds(r0 + j, 1)],
                                  ibuf.at[pl.ds(b, 1)], si[b]).wait()
            pltpu.sync_copy(mbuf.at[b], table.at[ibuf.at[b]], add=True)

        for b in range(2):
            start(b, b)

        def body(g, carry):
            j = 2 * g
            for b in range(2):
                proc(j + b, b)

                @pl.when(j + b + 2 < SC_NCH)
                def _():
                    start(j + b + 2, b)

            return carry

        lax.fori_loop(0, SC_NCH // 2, body, 0)
        plsc.subcore_barrier()
        last = HALF - 15 * TPS

        @pl.when(s < 15)
        def _():
            pltpu.sync_copy(table.at[pl.ds(s * TPS, TPS)],
                            out.at[pl.ds(c * HALF + s * TPS, TPS)])

        @pl.when(s == 15)
        def _():
            pltpu.sync_copy(table.at[pl.ds(15 * TPS, last)],
                            out.at[pl.ds(c * HALF + 15 * TPS, last)])

    return k


def _make_lin(interpret=False):
    def body(x_ref, w_ref, b_ref, o_ref):
        o_ref[...] = (
            jnp.dot(x_ref[...], w_ref[...], preferred_element_type=jnp.float32)
            + b_ref[...]
        )

    return pl.pallas_call(
        body,
        grid=(N // BN,),
        in_specs=[
            pl.BlockSpec((BN, 16), lambda i: (i, 0)),
            pl.BlockSpec((16, D), lambda i: (0, 0)),
            pl.BlockSpec((1, D), lambda i: (0, 0)),
        ],
        out_specs=pl.BlockSpec((BN, D), lambda i: (i, 0)),
        out_shape=jax.ShapeDtypeStruct((N, D), jnp.float32),
        interpret=interpret,
    )


def _make_proj(interpret=False):
    def body(h_ref, wd_ref, ws_ref, a_ref, b_ref):
        h = h_ref[...]
        a_ref[...] = jnp.dot(
            h, wd_ref[...], preferred_element_type=jnp.float32
        ).astype(jnp.bfloat16)
        b_ref[...] = jnp.dot(
            h, ws_ref[...], preferred_element_type=jnp.float32
        ).astype(jnp.bfloat16)

    return pl.pallas_call(
        body,
        grid=(N // BN,),
        in_specs=[
            pl.BlockSpec((BN, D), lambda i: (i, 0)),
            pl.BlockSpec((D, D), lambda i: (0, 0)),
            pl.BlockSpec((D, D), lambda i: (0, 0)),
        ],
        out_specs=[
            pl.BlockSpec((BN, D), lambda i: (i, 0)),
            pl.BlockSpec((BN, D), lambda i: (i, 0)),
        ],
        out_shape=[
            jax.ShapeDtypeStruct((N, D), jnp.bfloat16),
            jax.ShapeDtypeStruct((N, D), jnp.bfloat16),
        ],
        interpret=interpret,
    )


def _make_localidx(interpret=False):
    BR = 64

    def body(d_ref, o0_ref, o1_ref):
        i = pl.program_id(0)
        dv = d_ref[...]
        r2 = lax.broadcasted_iota(jnp.int32, (BR, CH), 0)
        c2 = lax.broadcasted_iota(jnp.int32, (BR, CH), 1)
        eid = (i * BR + r2) * CH + c2
        pad = eid >= E
        garb = GARB + (dv & 63)
        o0_ref[...] = jnp.where((dv < HALF) & ~pad, dv, garb)
        o1_ref[...] = jnp.where((dv >= HALF) & ~pad, dv - HALF, garb)

    return pl.pallas_call(
        body,
        grid=(ROWS2D // BR,),
        in_specs=[pl.BlockSpec((BR, CH), lambda i: (i, 0))],
        out_specs=[
            pl.BlockSpec((BR, CH), lambda i: (i, 0)),
            pl.BlockSpec((BR, CH), lambda i: (i, 0)),
        ],
        out_shape=[
            jax.ShapeDtypeStruct((ROWS2D, CH), jnp.int32),
            jax.ShapeDtypeStruct((ROWS2D, CH), jnp.int32),
        ],
        interpret=interpret,
    )


def _make_edge(interpret=False):
    def body(gd_ref, gs_ref, ea_ref, pd_ref, ps_ref, w1e_ref, w1p_ref,
             b1_ref, w2_ref, b2_ref, o_ref):
        dp = pd_ref[...] - ps_ref[...]
        d2 = jnp.sum(dp * dp, axis=1, keepdims=True)
        z = (
            gd_ref[...].astype(jnp.float32)
            + gs_ref[...].astype(jnp.float32)
            + jnp.dot(ea_ref[...], w1e_ref[...],
                      preferred_element_type=jnp.float32)
            + d2 * w1p_ref[...]
            + b1_ref[...]
        )
        m = (
            jnp.dot(jnp.maximum(z, 0.0), w2_ref[...],
                    preferred_element_type=jnp.float32)
            + b2_ref[...]
        )
        o_ref[...] = m

    return pl.pallas_call(
        body,
        grid=(EPAD // BE,),
        in_specs=[
            pl.BlockSpec((BE, D), lambda i: (i, 0)),
            pl.BlockSpec((BE, D), lambda i: (i, 0)),
            pl.BlockSpec((BE, 16), lambda i: (i, 0)),
            pl.BlockSpec((BE, 16), lambda i: (i, 0)),
            pl.BlockSpec((BE, 16), lambda i: (i, 0)),
            pl.BlockSpec((16, D), lambda i: (0, 0)),
            pl.BlockSpec((1, D), lambda i: (0, 0)),
            pl.BlockSpec((1, D), lambda i: (0, 0)),
            pl.BlockSpec((D, D), lambda i: (0, 0)),
            pl.BlockSpec((1, D), lambda i: (0, 0)),
        ],
        out_specs=pl.BlockSpec((BE, D), lambda i: (i, 0)),
        out_shape=jax.ShapeDtypeStruct((EPAD, D), jnp.float32),
        interpret=interpret,
    )


def _make_update(interpret=False):
    def body(h_ref, a_ref, u1h_ref, u1a_ref, ub1_ref, u2_ref, ub2_ref, o_ref):
        u = (
            jnp.dot(h_ref[...], u1h_ref[...],
                    preferred_element_type=jnp.float32)
            + jnp.dot(a_ref[...], u1a_ref[...],
                      preferred_element_type=jnp.float32)
            + ub1_ref[...]
        )
        o_ref[...] = (
            jnp.dot(jnp.maximum(u, 0.0), u2_ref[...],
                    preferred_element_type=jnp.float32)
            + ub2_ref[...]
        )

    return pl.pallas_call(
        body,
        grid=(N // BN,),
        in_specs=[
            pl.BlockSpec((BN, D), lambda i: (i, 0)),
            pl.BlockSpec((BN, D), lambda i: (i, 0)),
            pl.BlockSpec((D, D), lambda i: (0, 0)),
            pl.BlockSpec((D, D), lambda i: (0, 0)),
            pl.BlockSpec((1, D), lambda i: (0, 0)),
            pl.BlockSpec((D, D), lambda i: (0, 0)),
            pl.BlockSpec((1, D), lambda i: (0, 0)),
        ],
        out_specs=pl.BlockSpec((BN, D), lambda i: (i, 0)),
        out_shape=jax.ShapeDtypeStruct((N, D), jnp.float32),
        interpret=interpret,
    )


def _make_pool(interpret=False):
    nblk = N // BN

    def body(h_ref, b_ref, wh_ref, bh_ref, o_ref, acc, cnt):
        i = pl.program_id(0)

        @pl.when(i == 0)
        def _():
            acc[...] = jnp.zeros_like(acc)
            cnt[...] = jnp.zeros_like(cnt)

        bt = b_ref[...].reshape(1, BN)
        gi = lax.broadcasted_iota(jnp.int32, (G, BN), 0)
        oh = (bt == gi).astype(jnp.float32)
        acc[...] += jnp.dot(oh, h_ref[...], preferred_element_type=jnp.float32)
        cnt[...] += jnp.sum(oh, axis=1, keepdims=True)

        @pl.when(i == nblk - 1)
        def _():
            pooled = acc[...] / jnp.maximum(cnt[...], 1.0)
            o_ref[...] = (
                jnp.dot(pooled, wh_ref[...], preferred_element_type=jnp.float32)
                + bh_ref[...]
            )

    return pl.pallas_call(
        body,
        grid=(nblk,),
        in_specs=[
            pl.BlockSpec((BN, D), lambda i: (i, 0)),
            pl.BlockSpec((1, 1, BN), lambda i: (i, 0, 0)),
            pl.BlockSpec((D, 2 * 300), lambda i: (0, 0)),
            pl.BlockSpec((1, 2 * 300), lambda i: (0, 0)),
        ],
        out_specs=pl.BlockSpec((G, 2 * 300), lambda i: (0, 0)),
        out_shape=jax.ShapeDtypeStruct((G, 2 * 300), jnp.float32),
        scratch_shapes=[
            pltpu.VMEM((G, D), jnp.float32),
            pltpu.VMEM((G, 1), jnp.float32),
        ],
        interpret=interpret,
    )


def _build(interpret=False, sc_interpret=False):
    fns = {
        "lin": _make_lin(interpret),
        "proj": _make_proj(interpret),
        "localidx": _make_localidx(interpret),
        "edge": _make_edge(interpret),
        "update": _make_update(interpret),
        "pool": _make_pool(interpret),
        "gather16": _make_gather_pair(16, jnp.float32, sc_interpret),
        "gather64": _make_gather_pair(D, jnp.bfloat16, sc_interpret),
        "scatter": _make_scatter(sc_interpret),
    }
    return fns


@functools.lru_cache(maxsize=1)
def _default_fns():
    return _build()


def kernel(x, pos, edge_index, edge_attr, batch, W_in, b_in,
           msg_W1, msg_b1, msg_W2, msg_b2,
           upd_W1, upd_b1, upd_W2, upd_b2,
           head_e_W, head_e_b, head_i_W, head_i_b, _fns=None):
    f = _fns if _fns is not None else _default_fns()
    x16 = jnp.pad(x, ((0, 0), (0, 16 - x.shape[1])))
    win16 = jnp.pad(W_in, ((0, 16 - W_in.shape[0]), (0, 0)))
    pos16 = jnp.pad(pos, ((0, 0), (0, 16 - pos.shape[1])))
    src = jnp.pad(edge_index[0], (0, EPAD - E)).reshape(ROWS2D, CH)
    dst = jnp.pad(edge_index[1], (0, EPAD - E)).reshape(ROWS2D, CH)
    ea = jnp.pad(edge_attr, ((0, EPAD - E), (0, 0)))
    zrows = jnp.zeros((TPS, D), jnp.float32)
    batch3d = batch.reshape(N // BN, 1, BN)
    wh = jnp.concatenate([head_e_W, head_i_W], axis=1)
    bh = jnp.concatenate([head_e_b, head_i_b]).reshape(1, -1)

    h = f["lin"](x16, win16, b_in.reshape(1, D))
    pd, ps = f["gather16"](pos16, pos16, dst, src)
    idx0, idx1 = f["localidx"](dst)
    for l in range(NLAYERS):
        w1 = msg_W1[l]
        a, b = f["proj"](h, w1[:D], w1[D:2 * D])
        gd, gs = f["gather64"](a, b, dst, src)
        m = f["edge"](gd, gs, ea, pd, ps, w1[2 * D:2 * D + 16],
                      w1[2 * D + 16:], msg_b1[l].reshape(1, D),
                      msg_W2[l], msg_b2[l].reshape(1, D))
        agg = f["scatter"](m, idx0, idx1, zrows)
        h = f["update"](h, agg, upd_W1[l][:D], upd_W1[l][D:],
                        upd_b1[l].reshape(1, D), upd_W2[l],
                        upd_b2[l].reshape(1, D))
    return f["pool"](h, batch3d, wh, bh)
